# baseline (speedup 1.0000x reference)
import functools

import jax
import jax.numpy as jnp
from jax import lax
from jax.experimental import pallas as pl
from jax.experimental.pallas import tpu as pltpu
from jax.experimental.pallas import tpu_sc as plsc

BATCH = 16384
DIM = 128
LANES = 16
VPR = DIM // LANES
NC = 2
NS = 16
NW = NC * NS
CHUNK = 64
B_PER_W = BATCH // NW
NCHUNK = B_PER_W // CHUNK
RING = 3


def _sc_body(idx_t_hbm, idx_c_hbm, ww_hbm, wc_hbm, dw_hbm, db_hbm, out_hbm,
             *scratch):
    idx_t_v, idx_c_v = scratch[0], scratch[1]
    ring_bufs = scratch[2:2 + 2 * RING]
    out_v, dw_v, db_v = scratch[2 + 2 * RING:5 + 2 * RING]
    ring_sems = scratch[5 + 2 * RING:5 + 4 * RING]
    sem_o = scratch[5 + 4 * RING]
    bufs = tuple(
        (ring_bufs[2 * i], ring_bufs[2 * i + 1],
         ring_sems[2 * i], ring_sems[2 * i + 1]) for i in range(RING))
    wid = lax.axis_index("s") * NC + lax.axis_index("c")

    h_it = pltpu.async_copy(
        idx_t_hbm.at[pl.ds(wid * B_PER_W, B_PER_W)], idx_t_v, sem_o)
    h_ic = pltpu.async_copy(
        idx_c_hbm.at[pl.ds(wid * B_PER_W, B_PER_W)], idx_c_v, sem_o)
    h_dw = pltpu.async_copy(dw_hbm, dw_v, sem_o)
    h_db = pltpu.async_copy(db_hbm, db_v, sem_o)
    h_it.wait()
    h_ic.wait()
    h_dw.wait()
    h_db.wait()

    lane = lax.iota(jnp.int32, LANES)
    zero16 = jnp.zeros((LANES,), jnp.int32)
    dw = plsc.load_gather(dw_v, [zero16, zero16])
    db = plsc.load_gather(db_v, [zero16])

    def fire(j):
        wb, cb, sw, sc_ = bufs[j % RING]
        hw = pltpu.async_copy(
            ww_hbm.at[idx_t_v.at[pl.ds(j * CHUNK, CHUNK)]],
            wb.at[:, pl.ds(0, DIM)], sw)
        hc = pltpu.async_copy(
            wc_hbm.at[idx_c_v.at[pl.ds(j * CHUNK, CHUNK)]],
            cb.at[:, pl.ds(0, DIM)], sc_)
        return hw, hc

    handles = [fire(j) for j in range(RING - 1)]
    for j in range(NCHUNK):
        if j + RING - 1 < NCHUNK:
            handles.append(fire(j + RING - 1))
        hw, hc = handles[j]
        hw.wait()
        hc.wait()
        wb, cb, _, _ = bufs[j % RING]

        def group_body(g, _, j=j, wb=wb, cb=cb):
            rows = g * LANES + lane

            def col_body(k, accs):
                a0, a1, a2, a3 = accs
                c0 = zero16 + 4 * k
                c1 = zero16 + (4 * k + 1)
                c2 = zero16 + (4 * k + 2)
                c3 = zero16 + (4 * k + 3)
                a0 = a0 + (plsc.load_gather(wb, [rows, c0]) *
                           plsc.load_gather(cb, [rows, c0]))
                a1 = a1 + (plsc.load_gather(wb, [rows, c1]) *
                           plsc.load_gather(cb, [rows, c1]))
                a2 = a2 + (plsc.load_gather(wb, [rows, c2]) *
                           plsc.load_gather(cb, [rows, c2]))
                a3 = a3 + (plsc.load_gather(wb, [rows, c3]) *
                           plsc.load_gather(cb, [rows, c3]))
                return a0, a1, a2, a3

            zv = jnp.zeros((LANES,), jnp.float32)
            a0, a1, a2, a3 = lax.fori_loop(0, DIM // 4, col_body,
                                           (zv, zv, zv, zv), unroll=8)
            acc = (a0 + a1) + (a2 + a3)
            z = acc * dw + db
            out_v[pl.ds(j * CHUNK + g * LANES, LANES)] = (
                1.0 / (1.0 + jnp.exp(-z)))
            return _

        lax.fori_loop(0, CHUNK // LANES, group_body, None)

    pltpu.async_copy(out_v, out_hbm.at[pl.ds(wid * B_PER_W, B_PER_W)],
                     sem_o).wait()


@jax.jit
def _sc_call(idx_t, idx_c, W_word, W_ctx, dense_w, dense_b):
    mesh = plsc.VectorSubcoreMesh(core_axis_name="c", subcore_axis_name="s")
    f = functools.partial(
        pl.kernel,
        mesh=mesh,
        out_type=jax.ShapeDtypeStruct((BATCH,), jnp.float32),
        compiler_params=pltpu.CompilerParams(
            needs_layout_passes=False,
            disable_bounds_checks=True,
            disable_semaphore_checks=True,
            skip_device_barrier=True,
        ),
        scratch_types=[
            pltpu.VMEM((B_PER_W,), jnp.int32),
            pltpu.VMEM((B_PER_W,), jnp.int32),
        ] + [pltpu.VMEM((CHUNK, DIM + 1), jnp.float32)] * (2 * RING) + [
            pltpu.VMEM((B_PER_W,), jnp.float32),
            pltpu.VMEM((1, 1), jnp.float32),
            pltpu.VMEM((1,), jnp.float32),
        ] + [pltpu.SemaphoreType.DMA] * (2 * RING + 1),
    )(_sc_body)
    return f(idx_t, idx_c, W_word, W_ctx, dense_w, dense_b)


def kernel(word_target, word_context, W_word, W_ctx, dense_w, dense_b):
    out = _sc_call(word_target.reshape(-1), word_context.reshape(-1),
                   W_word, W_ctx, dense_w, dense_b)
    return out.reshape(BATCH, 1)


# speedup vs baseline: 2.7271x; 2.7271x over previous
import functools

import jax
import jax.numpy as jnp
from jax import lax
from jax.experimental import pallas as pl
from jax.experimental.pallas import tpu as pltpu
from jax.experimental.pallas import tpu_sc as plsc

BATCH = 16384
DIM = 128
LANES = 16
VPR = DIM // LANES
NC = 2
NS = 16
NW = NC * NS
CHUNK = 64
B_PER_W = BATCH // NW
NCHUNK = B_PER_W // CHUNK
RING = 3


def _sc_body(idx_t_hbm, idx_c_hbm, ww_hbm, wc_hbm, dw_hbm, db_hbm, out_hbm,
             *scratch):
    idx_t_v, idx_c_v = scratch[0], scratch[1]
    ring_bufs = scratch[2:2 + 2 * RING]
    out_v, dw_v, db_v = scratch[2 + 2 * RING:5 + 2 * RING]
    ring_sems = scratch[5 + 2 * RING:5 + 4 * RING]
    sem_o = scratch[5 + 4 * RING]
    bufs = tuple(
        (ring_bufs[2 * i], ring_bufs[2 * i + 1],
         ring_sems[2 * i], ring_sems[2 * i + 1]) for i in range(RING))
    wid = lax.axis_index("s") * NC + lax.axis_index("c")

    h_it = pltpu.async_copy(
        idx_t_hbm.at[pl.ds(wid * B_PER_W, B_PER_W)], idx_t_v, sem_o)
    h_ic = pltpu.async_copy(
        idx_c_hbm.at[pl.ds(wid * B_PER_W, B_PER_W)], idx_c_v, sem_o)
    h_dw = pltpu.async_copy(dw_hbm, dw_v, sem_o)
    h_db = pltpu.async_copy(db_hbm, db_v, sem_o)
    h_it.wait()
    h_ic.wait()
    h_dw.wait()
    h_db.wait()

    lane = lax.iota(jnp.int32, LANES)
    zero16 = jnp.zeros((LANES,), jnp.int32)
    dw = plsc.load_gather(dw_v, [zero16, zero16])
    db = plsc.load_gather(db_v, [zero16])

    def fire(j):
        wb, cb, sw, sc_ = bufs[j % RING]
        hw = pltpu.async_copy(
            ww_hbm.at[idx_t_v.at[pl.ds(j * CHUNK, CHUNK)]], wb, sw)
        hc = pltpu.async_copy(
            wc_hbm.at[idx_c_v.at[pl.ds(j * CHUNK, CHUNK)]], cb, sc_)
        return hw, hc

    handles = [fire(j) for j in range(RING - 1)]
    for j in range(NCHUNK):
        if j + RING - 1 < NCHUNK:
            handles.append(fire(j + RING - 1))
        hw, hc = handles[j]
        hw.wait()
        hc.wait()
        wb, cb, _, _ = bufs[j % RING]

        def group_body(g, _, j=j, wb=wb, cb=cb):
            rows = g * LANES + lane

            def col_body(k, accs):
                a0, a1, a2, a3 = accs
                c0 = (lane + 4 * k) & (DIM - 1)
                c1 = (lane + 4 * k + 1) & (DIM - 1)
                c2 = (lane + 4 * k + 2) & (DIM - 1)
                c3 = (lane + 4 * k + 3) & (DIM - 1)
                a0 = a0 + (plsc.load_gather(wb, [rows, c0]) *
                           plsc.load_gather(cb, [rows, c0]))
                a1 = a1 + (plsc.load_gather(wb, [rows, c1]) *
                           plsc.load_gather(cb, [rows, c1]))
                a2 = a2 + (plsc.load_gather(wb, [rows, c2]) *
                           plsc.load_gather(cb, [rows, c2]))
                a3 = a3 + (plsc.load_gather(wb, [rows, c3]) *
                           plsc.load_gather(cb, [rows, c3]))
                return a0, a1, a2, a3

            zv = jnp.zeros((LANES,), jnp.float32)
            a0, a1, a2, a3 = lax.fori_loop(0, DIM // 4, col_body,
                                           (zv, zv, zv, zv), unroll=8)
            acc = (a0 + a1) + (a2 + a3)
            z = acc * dw + db
            out_v[pl.ds(j * CHUNK + g * LANES, LANES)] = (
                1.0 / (1.0 + jnp.exp(-z)))
            return _

        lax.fori_loop(0, CHUNK // LANES, group_body, None)

    pltpu.async_copy(out_v, out_hbm.at[pl.ds(wid * B_PER_W, B_PER_W)],
                     sem_o).wait()


@jax.jit
def _sc_call(idx_t, idx_c, W_word, W_ctx, dense_w, dense_b):
    mesh = plsc.VectorSubcoreMesh(core_axis_name="c", subcore_axis_name="s")
    f = functools.partial(
        pl.kernel,
        mesh=mesh,
        out_type=jax.ShapeDtypeStruct((BATCH,), jnp.float32),
        compiler_params=pltpu.CompilerParams(
            needs_layout_passes=False,
            disable_bounds_checks=True,
            disable_semaphore_checks=True,
            skip_device_barrier=True,
        ),
        scratch_types=[
            pltpu.VMEM((B_PER_W,), jnp.int32),
            pltpu.VMEM((B_PER_W,), jnp.int32),
        ] + [pltpu.VMEM((CHUNK, DIM), jnp.float32)] * (2 * RING) + [
            pltpu.VMEM((B_PER_W,), jnp.float32),
            pltpu.VMEM((1, 1), jnp.float32),
            pltpu.VMEM((1,), jnp.float32),
        ] + [pltpu.SemaphoreType.DMA] * (2 * RING + 1),
    )(_sc_body)
    return f(idx_t, idx_c, W_word, W_ctx, dense_w, dense_b)


def kernel(word_target, word_context, W_word, W_ctx, dense_w, dense_b):
    out = _sc_call(word_target.reshape(-1), word_context.reshape(-1),
                   W_word, W_ctx, dense_w, dense_b)
    return out.reshape(BATCH, 1)
